# baseline (speedup 1.0000x reference)
import functools

import jax
import jax.numpy as jnp
from jax import lax
from jax.experimental import pallas as pl
from jax.experimental.pallas import tpu as pltpu
from jax.experimental.pallas import tpu_sc as plsc

N = 32768
D = 4096
G = 256

R_SC = 12288

NC = 2
NS = 16
NW = NC * NS
CW = D // NW
CH = 128
SPAN = 64
NCH = R_SC // CH
SUP = 2048
CPS = SUP // CH
NSUP = R_SC // SUP

BLK = 512
OFF = R_SC // BLK
NB_TC = (N - R_SC) // BLK


def _sc_pool_body(x_hbm, batch_hbm, maskf_hbm, sums_hbm, cnt_hbm,
                  batch_s, mask_s, buf0, buf1, acc, cnt16, regs_ref,
                  prev_ref, sem0, sem1):
    cid = lax.axis_index("c")
    sid = lax.axis_index("s")
    w = sid * NC + cid
    c0 = w * CW
    bufs = (buf0, buf1)
    sems = (sem0, sem1)

    z16 = jnp.zeros((16,), jnp.float32)

    def _zacc(g, c):
        for j in range(CW // 16):
            acc[g, pl.ds(j * 16, 16)] = z16
        cnt16[g, :] = z16
        return c
    lax.fori_loop(0, G, _zacc, 0)

    def _stage_start(ci, k):
        pltpu.async_copy(
            x_hbm.at[pl.ds(ci * CH, CH), pl.ds(c0, CW)], bufs[k], sems[k])

    def _stage_wait(ci, k):
        pltpu.make_async_copy(
            x_hbm.at[pl.ds(ci * CH, CH), pl.ds(c0, CW)], bufs[k], sems[k]).wait()

    NR = CW // 16

    def _span(b, u, sp):
        regs = [regs_ref[j, :] for j in range(NR)]
        creg = regs_ref[NR, :]
        prev = prev_ref[0]
        base = u * CH + sp * SPAN
        for g in range(SPAN // 16):
            b16 = batch_s[pl.ds(base + g * 16, 16)]
            m16 = mask_s[pl.ds(base + g * 16, 16)]
            rr0 = sp * SPAN + g * 16
            for i in range(16):
                seg = b16[i]
                mv = m16[i]
                keep = jnp.where(seg == prev, jnp.float32(1.0),
                                 jnp.float32(0.0))
                k16 = jax.lax.broadcast(keep, (16,))
                for j in range(NR):
                    v = b[rr0 + i, pl.ds(j * 16, 16)]
                    regs[j] = k16 * regs[j] + mv * v
                    acc[seg, pl.ds(j * 16, 16)] = regs[j]
                cmv = jax.lax.broadcast(mv, (16,))
                creg = k16 * creg + cmv
                cnt16[seg, :] = creg
                prev = seg
        for j in range(NR):
            regs_ref[j, :] = regs[j]
        regs_ref[NR, :] = creg
        prev_ref[0] = prev

    def _chunk(ci, k):
        _stage_wait(ci, k)
        _stage_start(jnp.minimum(ci + 1, NCH - 1), 1 - k)
        u = lax.rem(ci, CPS)

        def _sl(sp, c):
            _span(bufs[k], u, sp)
            return c
        lax.fori_loop(0, CH // SPAN, _sl, 0)

    def _load_super(s):
        pltpu.sync_copy(batch_hbm.at[pl.ds(s * SUP, SUP)], batch_s)
        pltpu.sync_copy(maskf_hbm.at[pl.ds(s * SUP, SUP)], mask_s)

    _stage_start(0, 0)
    for j in range(NR + 1):
        regs_ref[j, :] = z16
    prev_ref[0] = -1

    def _super(s, c):
        _load_super(s)

        def _pair(t, c2):
            ci = s * CPS + 2 * t
            _chunk(ci, 0)
            _chunk(ci + 1, 1)
            return c2
        lax.fori_loop(0, CPS // 2, _pair, 0)
        return c
    lax.fori_loop(0, NSUP, _super, 0)
    _stage_wait(NCH - 1, 0)

    pltpu.sync_copy(acc, sums_hbm.at[pl.ds(0, G), pl.ds(c0, CW)])
    pltpu.sync_copy(cnt16.at[pl.ds(w * (G // NW), G // NW)],
                    cnt_hbm.at[pl.ds(w * (G // NW), G // NW)])


_sc_pool = functools.partial(
    pl.kernel,
    out_type=(
        jax.ShapeDtypeStruct((G, D), jnp.float32),
        jax.ShapeDtypeStruct((G, 16), jnp.float32),
    ),
    mesh=plsc.VectorSubcoreMesh(core_axis_name="c", subcore_axis_name="s"),
    scratch_types=[
        pltpu.VMEM((SUP,), jnp.int32),
        pltpu.VMEM((SUP,), jnp.float32),
        pltpu.VMEM((CH, CW), jnp.float32),
        pltpu.VMEM((CH, CW), jnp.float32),
        pltpu.VMEM((G, CW), jnp.float32),
        pltpu.VMEM((G, 16), jnp.float32),
        pltpu.VMEM((D // NW // 16 + 1, 16), jnp.float32),
        pltpu.SMEM((1,), jnp.int32),
        pltpu.SemaphoreType.DMA,
        pltpu.SemaphoreType.DMA,
    ],
)(_sc_pool_body)


def _tc_pool_body(bb, mb, xb, sums, cnt, acc, cacc):
    i = pl.program_id(0)

    @pl.when(i == 0)
    def _init():
        acc[...] = jnp.zeros_like(acc)
        cacc[...] = jnp.zeros_like(cacc)

    b = bb[0]
    m = mb[0]
    seg_ids = jax.lax.broadcasted_iota(jnp.int32, (G, BLK), 0)
    oh = jnp.where(b == seg_ids, m, 0.0)
    acc[...] += jnp.dot(oh.astype(jnp.bfloat16), xb[...].astype(jnp.bfloat16),
                        preferred_element_type=jnp.float32)
    cacc[...] += jnp.sum(oh, axis=1, keepdims=True)

    @pl.when(i == NB_TC - 1)
    def _out():
        sums[...] = acc[...]
        cnt[...] = cacc[...]


def _head_body(s_tc, c_tc, s_sc, c_sc, wall, w1t, b1r, w2t, b2r, w3t, b3r, out):
    sums = s_tc[...] + s_sc[...]
    cnt = c_tc[...] + c_sc[:, :1]
    pooled = sums / jnp.maximum(cnt, 1.0)
    no = jnp.dot(pooled, wall[...], preferred_element_type=jnp.float32)
    h = jnp.maximum(jnp.dot(no, w1t[...], preferred_element_type=jnp.float32) + b1r[...], 0.0)
    h = jnp.maximum(jnp.dot(h, w2t[...], preferred_element_type=jnp.float32) + b2r[...], 0.0)
    out[...] = jnp.dot(h, w3t[...], preferred_element_type=jnp.float32) + b3r[...]


def kernel(x, idx_mask, batch, Wp, W1, b1, W2, b2, W3, b3):
    maskf = idx_mask.astype(jnp.float32)
    s_sc, c_sc = _sc_pool(x, batch, maskf)

    batch_r = batch.reshape(N // BLK, 1, BLK)
    mask_r = maskf.reshape(N // BLK, 1, BLK)
    s_tc, c_tc = pl.pallas_call(
        _tc_pool_body,
        grid=(NB_TC,),
        in_specs=[
            pl.BlockSpec((1, 1, BLK), lambda i: (i + OFF, 0, 0)),
            pl.BlockSpec((1, 1, BLK), lambda i: (i + OFF, 0, 0)),
            pl.BlockSpec((BLK, D), lambda i: (i + OFF, 0)),
        ],
        out_specs=[
            pl.BlockSpec((G, D), lambda i: (0, 0)),
            pl.BlockSpec((G, 1), lambda i: (0, 0)),
        ],
        out_shape=[
            jax.ShapeDtypeStruct((G, D), jnp.float32),
            jax.ShapeDtypeStruct((G, 1), jnp.float32),
        ],
        scratch_shapes=[
            pltpu.VMEM((G, D), jnp.float32),
            pltpu.VMEM((G, 1), jnp.float32),
        ],
    )(batch_r, mask_r, x)

    wall = Wp.transpose(0, 2, 1).reshape(D, 1024)
    w1t, w2t, w3t = W1.T, W2.T, W3.T
    b1r, b2r, b3r = b1.reshape(1, -1), b2.reshape(1, -1), b3.reshape(1, -1)

    return pl.pallas_call(
        _head_body,
        out_shape=jax.ShapeDtypeStruct((G, 1195), jnp.float32),
    )(s_tc, c_tc, s_sc, c_sc, wall, w1t, b1r, w2t, b2r, w3t, b3r)


# speedup vs baseline: 5.5123x; 5.5123x over previous
import jax
import jax.numpy as jnp
from jax.experimental import pallas as pl
from jax.experimental.pallas import tpu as pltpu

N_NODES = 32768
D = 4096
G = 256
BLK = 512
NB = N_NODES // BLK
NB2 = NB // 2


def _pool_body(ba, ma, bb_, mb, xa, xb, sums, cnt, acc, cacc):
    i = pl.program_id(0)

    @pl.when(i == 0)
    def _init():
        acc[...] = jnp.zeros_like(acc)
        cacc[...] = jnp.zeros_like(cacc)

    seg_ids = jax.lax.broadcasted_iota(jnp.int32, (G, BLK), 0)
    oha = jnp.where(ba[0] == seg_ids, ma[0], 0.0)
    ohb = jnp.where(bb_[0] == seg_ids, mb[0], 0.0)
    acc[...] += (
        jnp.dot(oha.astype(jnp.bfloat16), xa[...].astype(jnp.bfloat16),
                preferred_element_type=jnp.float32)
        + jnp.dot(ohb.astype(jnp.bfloat16), xb[...].astype(jnp.bfloat16),
                  preferred_element_type=jnp.float32))
    cacc[...] += (jnp.sum(oha, axis=1, keepdims=True)
                  + jnp.sum(ohb, axis=1, keepdims=True))

    @pl.when(i == NB2 - 1)
    def _out():
        sums[...] = acc[...]
        cnt[...] = cacc[...]


def _head_body(sums, cnt, wall, w1t, b1r, w2t, b2r, w3t, b3r, out):
    pooled = sums[...] / jnp.maximum(cnt[...], 1.0)
    no = jnp.dot(pooled, wall[...], preferred_element_type=jnp.float32)
    h = jnp.maximum(jnp.dot(no, w1t[...], preferred_element_type=jnp.float32) + b1r[...], 0.0)
    h = jnp.maximum(jnp.dot(h, w2t[...], preferred_element_type=jnp.float32) + b2r[...], 0.0)
    out[...] = jnp.dot(h, w3t[...], preferred_element_type=jnp.float32) + b3r[...]


def kernel(x, idx_mask, batch, Wp, W1, b1, W2, b2, W3, b3):
    batch_r = batch.reshape(NB, 1, BLK)
    mask_r = idx_mask.astype(jnp.float32).reshape(NB, 1, BLK)

    sums, cnt = pl.pallas_call(
        _pool_body,
        grid=(NB2,),
        in_specs=[
            pl.BlockSpec((1, 1, BLK), lambda i: (i, 0, 0)),
            pl.BlockSpec((1, 1, BLK), lambda i: (i, 0, 0)),
            pl.BlockSpec((1, 1, BLK), lambda i: (i + NB2, 0, 0)),
            pl.BlockSpec((1, 1, BLK), lambda i: (i + NB2, 0, 0)),
            pl.BlockSpec((BLK, D), lambda i: (i, 0)),
            pl.BlockSpec((BLK, D), lambda i: (i + NB2, 0)),
        ],
        out_specs=[
            pl.BlockSpec((G, D), lambda i: (0, 0)),
            pl.BlockSpec((G, 1), lambda i: (0, 0)),
        ],
        out_shape=[
            jax.ShapeDtypeStruct((G, D), jnp.float32),
            jax.ShapeDtypeStruct((G, 1), jnp.float32),
        ],
        scratch_shapes=[
            pltpu.VMEM((G, D), jnp.float32),
            pltpu.VMEM((G, 1), jnp.float32),
        ],
    )(batch_r, mask_r, batch_r, mask_r, x, x)

    wall = Wp.transpose(0, 2, 1).reshape(D, 1024)
    w1t, w2t, w3t = W1.T, W2.T, W3.T
    b1r, b2r, b3r = b1.reshape(1, -1), b2.reshape(1, -1), b3.reshape(1, -1)

    return pl.pallas_call(
        _head_body,
        out_shape=jax.ShapeDtypeStruct((G, 1195), jnp.float32),
    )(sums, cnt, wall, w1t, b1r, w2t, b2r, w3t, b3r)
